# baseline (speedup 1.0000x reference)
import functools

import jax
import jax.numpy as jnp
from jax import lax
from jax.experimental import pallas as pl
from jax.experimental.pallas import tpu as pltpu
from jax.experimental.pallas import tpu_sc as plsc

N = 10000
E = 640000
EMB = 128
L = 5

NC = 2
NS = 16
NW = NC * NS
EPT = E // NW

NQ = 4
QW = EMB // NQ
EPS = E // NS
CH = 125
NCH = EPS // CH
NP = 10240
RPT = NP // NS

CCH = 800
NCCH = EPT // CCH
NCOL = 12

_mesh = plsc.VectorSubcoreMesh(
    core_axis_name="c", subcore_axis_name="s", num_cores=NC, num_subcores=NS
)

BN = 1000
GRID = N // BN


@functools.partial(
    pl.kernel,
    out_type=jax.ShapeDtypeStruct((NW, N * NCOL), jnp.float32),
    mesh=_mesh,
    scratch_types=[
        pltpu.VMEM((CCH,), jnp.int32),
        pltpu.VMEM((CCH,), jnp.int32),
        pltpu.VMEM((CCH,), jnp.int32),
        pltpu.VMEM((N * NCOL,), jnp.float32),
    ],
    compiler_params=pltpu.CompilerParams(needs_layout_passes=False),
)
def _count_kernel(dst_hbm, w0_hbm, w1_hbm, out_hbm, dv, av, bv, cnt):
    cid = lax.axis_index("c")
    sid = lax.axis_index("s")
    wid = cid * NS + sid
    base = wid * EPT

    zeros16 = jnp.zeros((16,), jnp.float32)

    def _zero(i, _):
        cnt[pl.ds(i * 16, 16)] = zeros16
        return _

    lax.fori_loop(0, (N * NCOL) // 16, _zero, None)

    ones16 = jnp.ones((16,), jnp.float32)

    def _chunk(c, _):
        off = base + c * CCH
        pltpu.sync_copy(dst_hbm.at[pl.ds(off, CCH)], dv)
        pltpu.sync_copy(w0_hbm.at[pl.ds(off, CCH)], av)
        pltpu.sync_copy(w1_hbm.at[pl.ds(off, CCH)], bv)

        def _step(j, _):
            d = dv[pl.ds(j * 16, 16)]
            a = av[pl.ds(j * 16, 16)]
            b = bv[pl.ds(j * 16, 16)]
            row = d * NCOL
            plsc.addupdate_scatter(cnt, [row + a], ones16)
            plsc.addupdate_scatter(cnt, [row + (b + 6)], ones16)
            return _

        lax.fori_loop(0, CCH // 16, _step, None)
        return _

    lax.fori_loop(0, NCCH, _chunk, None)
    pltpu.sync_copy(cnt, out_hbm.at[wid])


@functools.partial(
    pl.kernel,
    out_type=jax.ShapeDtypeStruct((NQ, NP, QW), jnp.float32),
    mesh=_mesh,
    scratch_types=[
        pltpu.VMEM((NCH, CH), jnp.int32),
        pltpu.VMEM((NCH, CH), jnp.int32),
        pltpu.VMEM((2, CH, QW), jnp.float32),
        pltpu.VMEM((RPT, QW), jnp.float32),
        pltpu.VMEM_SHARED((NP, QW), jnp.float32),
        pltpu.SemaphoreType.DMA((2,)),
        pltpu.SemaphoreType.DMA((2,)),
    ],
    compiler_params=pltpu.CompilerParams(
        needs_layout_passes=False, use_tc_tiling_on_sc=False
    ),
)
def _seg_kernel(src_hbm, dst_hbm, h_hbm, out_hbm, sidx, didx, rows, stage, agg,
                gsem, ssem):
    cid = lax.axis_index("c")
    sid = lax.axis_index("s")

    pltpu.sync_copy(src_hbm.at[sid], sidx)
    pltpu.sync_copy(dst_hbm.at[sid], didx)

    zeros16 = jnp.zeros((16,), jnp.float32)

    def _zrow(i, _):
        r = i // (QW // 16)
        k = i % (QW // 16)
        stage[r, pl.ds(k * 16, 16)] = zeros16
        return _

    for p in range(NQ // NC):
        hq = h_hbm.at[NC * p + cid]

        lax.fori_loop(0, RPT * (QW // 16), _zrow, None)
        pltpu.sync_copy(stage, agg.at[pl.ds(sid * RPT, RPT)])
        plsc.subcore_barrier()

        pltpu.async_copy(hq.at[sidx.at[0]], rows.at[0], gsem.at[0])

        def _chunk(i, _):
            b = lax.rem(i, 2)
            nb = lax.rem(i + 1, 2)
            pltpu.make_async_copy(hq.at[sidx.at[i]], rows.at[b], gsem.at[b]).wait()

            @pl.when(i >= 1)
            def _():
                pltpu.make_async_copy(
                    rows.at[nb], agg.at[didx.at[i - 1]], ssem.at[nb]
                ).wait()

            @pl.when(i + 1 < NCH)
            def _():
                pltpu.async_copy(hq.at[sidx.at[i + 1]], rows.at[nb], gsem.at[nb])

            pltpu.async_copy(rows.at[b], agg.at[didx.at[i]], ssem.at[b], add=True)
            return _

        lax.fori_loop(0, NCH, _chunk, None)
        lb = (NCH - 1) % 2
        pltpu.make_async_copy(
            rows.at[lb], agg.at[didx.at[NCH - 1]], ssem.at[lb]
        ).wait()
        plsc.subcore_barrier()

        pltpu.sync_copy(agg.at[pl.ds(sid * RPT, RPT)], stage)
        pltpu.sync_copy(stage, out_hbm.at[NC * p + cid, pl.ds(sid * RPT, RPT)])


def _reduce_body(cntp_ref, cnt_ref):
    cnt_ref[...] = jnp.sum(cntp_ref[...], axis=0)


_reduce_call = pl.pallas_call(
    _reduce_body,
    grid=(GRID,),
    in_specs=[pl.BlockSpec((NW, BN, NCOL), lambda i: (0, i, 0))],
    out_specs=pl.BlockSpec((BN, NCOL), lambda i: (i, 0)),
    out_shape=jax.ShapeDtypeStruct((N, NCOL), jnp.float32),
)


def _init_body(x0_ref, x1_ref, ne0_ref, ne1_ref, wp_ref, bp_ref, h0_ref, fin_ref):
    ir = lax.broadcasted_iota(jnp.int32, (BN, 128), 1)
    oh0 = (x0_ref[...] == ir).astype(jnp.float32)
    oh1 = (x1_ref[...] == ir).astype(jnp.float32)
    h0 = jnp.dot(oh0, ne0_ref[...], preferred_element_type=jnp.float32)
    h0 = h0 + jnp.dot(oh1, ne1_ref[...], preferred_element_type=jnp.float32)
    for q in range(NQ):
        h0_ref[q] = h0[:, q * QW:(q + 1) * QW]
    fin_ref[...] = jnp.dot(h0, wp_ref[...], preferred_element_type=jnp.float32) + bp_ref[...]


_init_call = pl.pallas_call(
    _init_body,
    grid=(GRID,),
    in_specs=[
        pl.BlockSpec((BN, 1), lambda i: (i, 0)),
        pl.BlockSpec((BN, 1), lambda i: (i, 0)),
        pl.BlockSpec((128, EMB), lambda i: (0, 0)),
        pl.BlockSpec((128, EMB), lambda i: (0, 0)),
        pl.BlockSpec((EMB, EMB), lambda i: (0, 0)),
        pl.BlockSpec((1, EMB), lambda i: (0, 0)),
    ],
    out_specs=[
        pl.BlockSpec((NQ, BN, QW), lambda i: (0, i, 0)),
        pl.BlockSpec((BN, EMB), lambda i: (i, 0)),
    ],
    out_shape=[
        jax.ShapeDtypeStruct((NQ, N, QW), jnp.float32),
        jax.ShapeDtypeStruct((N, EMB), jnp.float32),
    ],
)


def _make_mlp_call(last: bool):
    def _mlp_body(p_ref, cnt_ref, e12_ref, w1_ref, b1_ref, w2_ref, b2_ref,
                  ga_ref, be_ref, wp_ref, fin_ref, h_ref, fino_ref):
        agg = jnp.concatenate([p_ref[q] for q in range(NQ)], axis=1)
        agg = agg + jnp.dot(cnt_ref[...], e12_ref[...],
                            preferred_element_type=jnp.float32)
        hmid = jnp.dot(agg, w1_ref[...], preferred_element_type=jnp.float32)
        hmid = jnp.maximum(hmid + b1_ref[...], 0.0)
        h = jnp.dot(hmid, w2_ref[...], preferred_element_type=jnp.float32)
        h = ga_ref[...] * (h + b2_ref[...]) + be_ref[...]
        if not last:
            h = jnp.maximum(h, 0.0)
        for q in range(NQ):
            h_ref[q] = h[:, q * QW:(q + 1) * QW]
        fino_ref[...] = fin_ref[...] + jnp.dot(h, wp_ref[...],
                                               preferred_element_type=jnp.float32)

    return pl.pallas_call(
        _mlp_body,
        grid=(GRID,),
        in_specs=[
            pl.BlockSpec((NQ, BN, QW), lambda i: (0, i, 0)),
            pl.BlockSpec((BN, NCOL), lambda i: (i, 0)),
            pl.BlockSpec((NCOL, EMB), lambda i: (0, 0)),
            pl.BlockSpec((EMB, 2 * EMB), lambda i: (0, 0)),
            pl.BlockSpec((1, 2 * EMB), lambda i: (0, 0)),
            pl.BlockSpec((2 * EMB, EMB), lambda i: (0, 0)),
            pl.BlockSpec((1, EMB), lambda i: (0, 0)),
            pl.BlockSpec((1, EMB), lambda i: (0, 0)),
            pl.BlockSpec((1, EMB), lambda i: (0, 0)),
            pl.BlockSpec((EMB, EMB), lambda i: (0, 0)),
            pl.BlockSpec((BN, EMB), lambda i: (i, 0)),
        ],
        out_specs=[
            pl.BlockSpec((NQ, BN, QW), lambda i: (0, i, 0)),
            pl.BlockSpec((BN, EMB), lambda i: (i, 0)),
        ],
        out_shape=[
            jax.ShapeDtypeStruct((NQ, N, QW), jnp.float32),
            jax.ShapeDtypeStruct((N, EMB), jnp.float32),
        ],
    )


_mlp_mid = _make_mlp_call(last=False)
_mlp_last = _make_mlp_call(last=True)


def kernel(g, x, w, node_emb0, node_emb1, edge_emb0, edge_emb1,
           W1, b1, W2, b2, gamma, beta, Wp, bp):
    src = g[0].reshape(NS, NCH, CH)
    dst = g[1]
    wt = w.T
    w0 = wt[0]
    w1 = wt[1]
    dst3 = dst.reshape(NS, NCH, CH)

    ne0 = jnp.zeros((128, EMB), jnp.float32).at[: node_emb0.shape[0]].set(node_emb0)
    ne1 = jnp.zeros((128, EMB), jnp.float32).at[: node_emb1.shape[0]].set(node_emb1)

    e12 = jnp.zeros((L, NCOL, EMB), jnp.float32)
    e12 = e12.at[:, 0:6].set(edge_emb0)
    e12 = e12.at[:, 6:9].set(edge_emb1)

    wp = Wp.reshape(L + 1, EMB, EMB)

    cntp = _count_kernel(dst, w0, w1)
    cnt = _reduce_call(cntp.reshape(NW, N, NCOL))

    h, fin = _init_call(x[:, 0:1], x[:, 1:2], ne0, ne1, wp[0], bp.reshape(1, EMB))

    for l in range(L):
        p = _seg_kernel(src, dst3, h)
        call = _mlp_last if l == L - 1 else _mlp_mid
        h, fin = call(
            p, cnt, e12[l],
            W1[l], b1[l].reshape(1, 2 * EMB),
            W2[l], b2[l].reshape(1, EMB),
            gamma[l].reshape(1, EMB), beta[l].reshape(1, EMB),
            wp[l + 1], fin,
        )
    return fin


# speedup vs baseline: 1.6867x; 1.6867x over previous
import functools

import jax
import jax.numpy as jnp
from jax import lax
from jax.experimental import pallas as pl
from jax.experimental.pallas import tpu as pltpu
from jax.experimental.pallas import tpu_sc as plsc

N = 10000
E = 640000
EMB = 128
L = 5

NC = 2
NS = 16
NW = NC * NS
EPT = E // NW

NQ = 4
QW = EMB // NQ
EPS = E // NS
CH = 1000
NCH = EPS // CH
NP = 10240
RPT = NP // NS
SROW = 320

CCH = 800
NCCH = EPT // CCH
NCOL = 12

_mesh = plsc.VectorSubcoreMesh(
    core_axis_name="c", subcore_axis_name="s", num_cores=NC, num_subcores=NS
)

BN = 1000
GRID = N // BN


@functools.partial(
    pl.kernel,
    out_type=jax.ShapeDtypeStruct((NW, N * NCOL), jnp.float32),
    mesh=_mesh,
    scratch_types=[
        pltpu.VMEM((CCH,), jnp.int32),
        pltpu.VMEM((CCH,), jnp.int32),
        pltpu.VMEM((CCH,), jnp.int32),
        pltpu.VMEM((N * NCOL,), jnp.float32),
    ],
    compiler_params=pltpu.CompilerParams(needs_layout_passes=False),
)
def _count_kernel(dst_hbm, w0_hbm, w1_hbm, out_hbm, dv, av, bv, cnt):
    cid = lax.axis_index("c")
    sid = lax.axis_index("s")
    wid = cid * NS + sid
    base = wid * EPT

    zeros16 = jnp.zeros((16,), jnp.float32)

    def _zero(i, _):
        cnt[pl.ds(i * 16, 16)] = zeros16
        return _

    lax.fori_loop(0, (N * NCOL) // 16, _zero, None)

    ones16 = jnp.ones((16,), jnp.float32)

    def _chunk(c, _):
        off = base + c * CCH
        pltpu.sync_copy(dst_hbm.at[pl.ds(off, CCH)], dv)
        pltpu.sync_copy(w0_hbm.at[pl.ds(off, CCH)], av)
        pltpu.sync_copy(w1_hbm.at[pl.ds(off, CCH)], bv)

        def _step(j, _):
            d = dv[pl.ds(j * 16, 16)]
            a = av[pl.ds(j * 16, 16)]
            b = bv[pl.ds(j * 16, 16)]
            row = d * NCOL
            plsc.addupdate_scatter(cnt, [row + a], ones16)
            plsc.addupdate_scatter(cnt, [row + (b + 6)], ones16)
            return _

        lax.fori_loop(0, CCH // 16, _step, None)
        return _

    lax.fori_loop(0, NCCH, _chunk, None)
    pltpu.sync_copy(cnt, out_hbm.at[wid])


@functools.partial(
    pl.kernel,
    out_type=jax.ShapeDtypeStruct((NQ, NP, QW), jnp.float32),
    mesh=_mesh,
    scratch_types=[
        pltpu.VMEM((2, 2, CH), jnp.int32),
        pltpu.VMEM((2, CH, QW), jnp.float32),
        pltpu.VMEM((SROW, QW), jnp.float32),
        pltpu.VMEM_SHARED((NP, QW), jnp.float32),
        pltpu.SemaphoreType.DMA((2,)),
        pltpu.SemaphoreType.DMA((2,)),
    ],
    compiler_params=pltpu.CompilerParams(
        needs_layout_passes=False, use_tc_tiling_on_sc=False
    ),
)
def _seg_kernel(sd_hbm, h_hbm, out_hbm, cb, rows, stage, agg, gsem, ssem):
    cid = lax.axis_index("c")
    sid = lax.axis_index("s")

    zeros16 = jnp.zeros((16,), jnp.float32)

    def _zrow(i, _):
        r = i // (QW // 16)
        k = i % (QW // 16)
        stage[r, pl.ds(k * 16, 16)] = zeros16
        return _

    for p in range(NQ // NC):
        hq = h_hbm.at[NC * p + cid]

        lax.fori_loop(0, SROW * (QW // 16), _zrow, None)
        pltpu.sync_copy(stage, agg.at[pl.ds(sid * RPT, SROW)])
        pltpu.sync_copy(stage, agg.at[pl.ds(sid * RPT + SROW, SROW)])
        plsc.subcore_barrier()

        pltpu.sync_copy(sd_hbm.at[sid, 0], cb.at[0])
        pltpu.async_copy(hq.at[cb.at[0, 0]], rows.at[0], gsem.at[0])

        def _chunk(i, _):
            b = lax.rem(i, 2)
            nb = lax.rem(i + 1, 2)
            pltpu.make_async_copy(hq.at[cb.at[b, 0]], rows.at[b], gsem.at[b]).wait()

            @pl.when(i >= 1)
            def _():
                pltpu.make_async_copy(
                    rows.at[nb], agg.at[cb.at[nb, 1]], ssem.at[nb]
                ).wait()

            @pl.when(i + 1 < NCH)
            def _():
                pltpu.sync_copy(sd_hbm.at[sid, i + 1], cb.at[nb])
                pltpu.async_copy(hq.at[cb.at[nb, 0]], rows.at[nb], gsem.at[nb])

            pltpu.async_copy(rows.at[b], agg.at[cb.at[b, 1]], ssem.at[b], add=True)
            return _

        lax.fori_loop(0, NCH, _chunk, None)
        lb = (NCH - 1) % 2
        pltpu.make_async_copy(
            rows.at[lb], agg.at[cb.at[lb, 1]], ssem.at[lb]
        ).wait()
        plsc.subcore_barrier()

        pltpu.sync_copy(agg.at[pl.ds(sid * RPT, SROW)], stage)
        pltpu.sync_copy(stage, out_hbm.at[NC * p + cid, pl.ds(sid * RPT, SROW)])
        pltpu.sync_copy(agg.at[pl.ds(sid * RPT + SROW, SROW)], stage)
        pltpu.sync_copy(
            stage, out_hbm.at[NC * p + cid, pl.ds(sid * RPT + SROW, SROW)]
        )


def _reduce_body(cntp_ref, cnt_ref):
    cnt_ref[...] = jnp.sum(cntp_ref[...], axis=0)


_reduce_call = pl.pallas_call(
    _reduce_body,
    grid=(GRID,),
    in_specs=[pl.BlockSpec((NW, BN, NCOL), lambda i: (0, i, 0))],
    out_specs=pl.BlockSpec((BN, NCOL), lambda i: (i, 0)),
    out_shape=jax.ShapeDtypeStruct((N, NCOL), jnp.float32),
)


def _init_body(x0_ref, x1_ref, ne0_ref, ne1_ref, wp_ref, bp_ref, h0_ref, fin_ref):
    ir = lax.broadcasted_iota(jnp.int32, (BN, 128), 1)
    oh0 = (x0_ref[...] == ir).astype(jnp.float32)
    oh1 = (x1_ref[...] == ir).astype(jnp.float32)
    h0 = jnp.dot(oh0, ne0_ref[...], preferred_element_type=jnp.float32)
    h0 = h0 + jnp.dot(oh1, ne1_ref[...], preferred_element_type=jnp.float32)
    for q in range(NQ):
        h0_ref[q] = h0[:, q * QW:(q + 1) * QW]
    fin_ref[...] = jnp.dot(h0, wp_ref[...], preferred_element_type=jnp.float32) + bp_ref[...]


_init_call = pl.pallas_call(
    _init_body,
    grid=(GRID,),
    in_specs=[
        pl.BlockSpec((BN, 1), lambda i: (i, 0)),
        pl.BlockSpec((BN, 1), lambda i: (i, 0)),
        pl.BlockSpec((128, EMB), lambda i: (0, 0)),
        pl.BlockSpec((128, EMB), lambda i: (0, 0)),
        pl.BlockSpec((EMB, EMB), lambda i: (0, 0)),
        pl.BlockSpec((1, EMB), lambda i: (0, 0)),
    ],
    out_specs=[
        pl.BlockSpec((NQ, BN, QW), lambda i: (0, i, 0)),
        pl.BlockSpec((BN, EMB), lambda i: (i, 0)),
    ],
    out_shape=[
        jax.ShapeDtypeStruct((NQ, N, QW), jnp.float32),
        jax.ShapeDtypeStruct((N, EMB), jnp.float32),
    ],
)


def _make_mlp_call(last: bool):
    def _mlp_body(p_ref, cnt_ref, e12_ref, w1_ref, b1_ref, w2_ref, b2_ref,
                  ga_ref, be_ref, wp_ref, fin_ref, h_ref, fino_ref):
        agg = jnp.concatenate([p_ref[q] for q in range(NQ)], axis=1)
        agg = agg + jnp.dot(cnt_ref[...], e12_ref[...],
                            preferred_element_type=jnp.float32)
        hmid = jnp.dot(agg, w1_ref[...], preferred_element_type=jnp.float32)
        hmid = jnp.maximum(hmid + b1_ref[...], 0.0)
        h = jnp.dot(hmid, w2_ref[...], preferred_element_type=jnp.float32)
        h = ga_ref[...] * (h + b2_ref[...]) + be_ref[...]
        if not last:
            h = jnp.maximum(h, 0.0)
        for q in range(NQ):
            h_ref[q] = h[:, q * QW:(q + 1) * QW]
        fino_ref[...] = fin_ref[...] + jnp.dot(h, wp_ref[...],
                                               preferred_element_type=jnp.float32)

    return pl.pallas_call(
        _mlp_body,
        grid=(GRID,),
        in_specs=[
            pl.BlockSpec((NQ, BN, QW), lambda i: (0, i, 0)),
            pl.BlockSpec((BN, NCOL), lambda i: (i, 0)),
            pl.BlockSpec((NCOL, EMB), lambda i: (0, 0)),
            pl.BlockSpec((EMB, 2 * EMB), lambda i: (0, 0)),
            pl.BlockSpec((1, 2 * EMB), lambda i: (0, 0)),
            pl.BlockSpec((2 * EMB, EMB), lambda i: (0, 0)),
            pl.BlockSpec((1, EMB), lambda i: (0, 0)),
            pl.BlockSpec((1, EMB), lambda i: (0, 0)),
            pl.BlockSpec((1, EMB), lambda i: (0, 0)),
            pl.BlockSpec((EMB, EMB), lambda i: (0, 0)),
            pl.BlockSpec((BN, EMB), lambda i: (i, 0)),
        ],
        out_specs=[
            pl.BlockSpec((NQ, BN, QW), lambda i: (0, i, 0)),
            pl.BlockSpec((BN, EMB), lambda i: (i, 0)),
        ],
        out_shape=[
            jax.ShapeDtypeStruct((NQ, N, QW), jnp.float32),
            jax.ShapeDtypeStruct((N, EMB), jnp.float32),
        ],
    )


_mlp_mid = _make_mlp_call(last=False)
_mlp_last = _make_mlp_call(last=True)


def kernel(g, x, w, node_emb0, node_emb1, edge_emb0, edge_emb1,
           W1, b1, W2, b2, gamma, beta, Wp, bp):
    dst = g[1]
    wt = w.T
    w0 = wt[0]
    w1 = wt[1]
    sd = jnp.stack(
        [g[0].reshape(NS, NCH, CH), dst.reshape(NS, NCH, CH)], axis=2
    )

    ne0 = jnp.zeros((128, EMB), jnp.float32).at[: node_emb0.shape[0]].set(node_emb0)
    ne1 = jnp.zeros((128, EMB), jnp.float32).at[: node_emb1.shape[0]].set(node_emb1)

    e12 = jnp.zeros((L, NCOL, EMB), jnp.float32)
    e12 = e12.at[:, 0:6].set(edge_emb0)
    e12 = e12.at[:, 6:9].set(edge_emb1)

    wp = Wp.reshape(L + 1, EMB, EMB)

    cntp = _count_kernel(dst, w0, w1)
    cnt = _reduce_call(cntp.reshape(NW, N, NCOL))

    h, fin = _init_call(x[:, 0:1], x[:, 1:2], ne0, ne1, wp[0], bp.reshape(1, EMB))

    for l in range(L):
        p = _seg_kernel(sd, h)
        call = _mlp_last if l == L - 1 else _mlp_mid
        h, fin = call(
            p, cnt, e12[l],
            W1[l], b1[l].reshape(1, 2 * EMB),
            W2[l], b2[l].reshape(1, EMB),
            gamma[l].reshape(1, EMB), beta[l].reshape(1, EMB),
            wp[l + 1], fin,
        )
    return fin


# speedup vs baseline: 2.1289x; 1.2622x over previous
import functools

import jax
import jax.numpy as jnp
from jax import lax
from jax.experimental import pallas as pl
from jax.experimental.pallas import tpu as pltpu
from jax.experimental.pallas import tpu_sc as plsc

N = 10000
E = 640000
EMB = 128
L = 5

NC = 2
NS = 16
NW = NC * NS
EPT = E // NW

NQ = 4
QW = EMB // NQ
EPS = E // NS
CH = 1000
NCH = EPS // CH
NP = 10240
RPT = NP // NS
SROW = 320

CCH = 800
NCCH = EPT // CCH
NCOL = 12

_mesh = plsc.VectorSubcoreMesh(
    core_axis_name="c", subcore_axis_name="s", num_cores=NC, num_subcores=NS
)

BN = 1280
GRID = NP // BN


@functools.partial(
    pl.kernel,
    out_type=jax.ShapeDtypeStruct((NW, NP * NCOL), jnp.float32),
    mesh=_mesh,
    scratch_types=[
        pltpu.VMEM((CCH,), jnp.int32),
        pltpu.VMEM((CCH,), jnp.int32),
        pltpu.VMEM((CCH,), jnp.int32),
        pltpu.VMEM((NP * NCOL,), jnp.float32),
    ],
    compiler_params=pltpu.CompilerParams(needs_layout_passes=False),
)
def _count_kernel(dst_hbm, w0_hbm, w1_hbm, out_hbm, dv, av, bv, cnt):
    cid = lax.axis_index("c")
    sid = lax.axis_index("s")
    wid = cid * NS + sid
    base = wid * EPT

    zeros16 = jnp.zeros((16,), jnp.float32)

    def _zero(i, _):
        cnt[pl.ds(i * 16, 16)] = zeros16
        return _

    lax.fori_loop(0, (NP * NCOL) // 16, _zero, None)

    ones16 = jnp.ones((16,), jnp.float32)

    def _chunk(c, _):
        off = base + c * CCH
        pltpu.sync_copy(dst_hbm.at[pl.ds(off, CCH)], dv)
        pltpu.sync_copy(w0_hbm.at[pl.ds(off, CCH)], av)
        pltpu.sync_copy(w1_hbm.at[pl.ds(off, CCH)], bv)

        def _step(j, _):
            d = dv[pl.ds(j * 16, 16)]
            a = av[pl.ds(j * 16, 16)]
            b = bv[pl.ds(j * 16, 16)]
            plsc.addupdate_scatter(cnt, [a * NP + d], ones16)
            plsc.addupdate_scatter(cnt, [(b + 6) * NP + d], ones16)
            return _

        lax.fori_loop(0, CCH // 16, _step, None)
        return _

    lax.fori_loop(0, NCCH, _chunk, None)
    pltpu.sync_copy(cnt, out_hbm.at[wid])


XCOL = 8


@functools.partial(
    pl.kernel,
    out_type=jax.ShapeDtypeStruct((NW, NP * XCOL), jnp.float32),
    mesh=_mesh,
    scratch_types=[
        pltpu.VMEM((CCH,), jnp.int32),
        pltpu.VMEM((CCH,), jnp.int32),
        pltpu.VMEM((N,), jnp.int32),
        pltpu.VMEM((NP * XCOL,), jnp.float32),
    ],
    compiler_params=pltpu.CompilerParams(needs_layout_passes=False),
)
def _countx_kernel(dst_hbm, src_hbm, xc_hbm, out_hbm, dv, sv, xct, cnt):
    cid = lax.axis_index("c")
    sid = lax.axis_index("s")
    wid = cid * NS + sid
    base = wid * EPT

    pltpu.sync_copy(xc_hbm, xct)

    zeros16 = jnp.zeros((16,), jnp.float32)

    def _zero(i, _):
        cnt[pl.ds(i * 16, 16)] = zeros16
        return _

    lax.fori_loop(0, (NP * XCOL) // 16, _zero, None)

    ones16 = jnp.ones((16,), jnp.float32)

    def _chunk(c, _):
        off = base + c * CCH
        pltpu.sync_copy(dst_hbm.at[pl.ds(off, CCH)], dv)
        pltpu.sync_copy(src_hbm.at[pl.ds(off, CCH)], sv)

        def _step(j, _):
            d = dv[pl.ds(j * 16, 16)]
            s = sv[pl.ds(j * 16, 16)]
            xcv = plsc.load_gather(xct, [s])
            x0 = xcv & 3
            x1 = xcv >> 2
            plsc.addupdate_scatter(cnt, [x0 * NP + d], ones16)
            plsc.addupdate_scatter(cnt, [(x1 + 4) * NP + d], ones16)
            return _

        lax.fori_loop(0, CCH // 16, _step, None)
        return _

    lax.fori_loop(0, NCCH, _chunk, None)
    pltpu.sync_copy(cnt, out_hbm.at[wid])


@functools.partial(
    pl.kernel,
    out_type=jax.ShapeDtypeStruct((NQ, NP, QW), jnp.float32),
    mesh=_mesh,
    scratch_types=[
        pltpu.VMEM((2, 2, CH), jnp.int32),
        pltpu.VMEM((2, CH, QW), jnp.float32),
        pltpu.VMEM((SROW, QW), jnp.float32),
        pltpu.VMEM_SHARED((NP, QW), jnp.float32),
        pltpu.SemaphoreType.DMA((2,)),
        pltpu.SemaphoreType.DMA((2,)),
    ],
    compiler_params=pltpu.CompilerParams(
        needs_layout_passes=False, use_tc_tiling_on_sc=False
    ),
)
def _seg_kernel(sd_hbm, h_hbm, out_hbm, cb, rows, stage, agg, gsem, ssem):
    cid = lax.axis_index("c")
    sid = lax.axis_index("s")

    zeros16 = jnp.zeros((16,), jnp.float32)

    def _zrow(i, _):
        r = i // (QW // 16)
        k = i % (QW // 16)
        stage[r, pl.ds(k * 16, 16)] = zeros16
        return _

    for p in range(NQ // NC):
        hq = h_hbm.at[NC * p + cid]

        lax.fori_loop(0, SROW * (QW // 16), _zrow, None)
        pltpu.sync_copy(stage, agg.at[pl.ds(sid * RPT, SROW)])
        pltpu.sync_copy(stage, agg.at[pl.ds(sid * RPT + SROW, SROW)])
        plsc.subcore_barrier()

        pltpu.sync_copy(sd_hbm.at[sid, 0], cb.at[0])
        pltpu.async_copy(hq.at[cb.at[0, 0]], rows.at[0], gsem.at[0])

        def _chunk(i, _):
            b = lax.rem(i, 2)
            nb = lax.rem(i + 1, 2)
            pltpu.make_async_copy(hq.at[cb.at[b, 0]], rows.at[b], gsem.at[b]).wait()

            @pl.when(i >= 1)
            def _():
                pltpu.make_async_copy(
                    rows.at[nb], agg.at[cb.at[nb, 1]], ssem.at[nb]
                ).wait()

            @pl.when(i + 1 < NCH)
            def _():
                pltpu.sync_copy(sd_hbm.at[sid, i + 1], cb.at[nb])
                pltpu.async_copy(hq.at[cb.at[nb, 0]], rows.at[nb], gsem.at[nb])

            pltpu.async_copy(rows.at[b], agg.at[cb.at[b, 1]], ssem.at[b], add=True)
            return _

        lax.fori_loop(0, NCH, _chunk, None)
        lb = (NCH - 1) % 2
        pltpu.make_async_copy(
            rows.at[lb], agg.at[cb.at[lb, 1]], ssem.at[lb]
        ).wait()
        plsc.subcore_barrier()

        pltpu.sync_copy(agg.at[pl.ds(sid * RPT, SROW)], stage)
        pltpu.sync_copy(stage, out_hbm.at[NC * p + cid, pl.ds(sid * RPT, SROW)])
        pltpu.sync_copy(agg.at[pl.ds(sid * RPT + SROW, SROW)], stage)
        pltpu.sync_copy(
            stage, out_hbm.at[NC * p + cid, pl.ds(sid * RPT + SROW, SROW)]
        )


def _cdot(cnt_t, tab):
    return lax.dot_general(
        cnt_t, tab, (((0,), (0,)), ((), ())),
        preferred_element_type=jnp.float32,
    )


def _prep_body(cntp_ref, cntxp_ref, x0_ref, x1_ref, ne0_ref, ne1_ref,
               wp0_ref, bp_ref, e12_ref, ex8_ref, w1_ref, b1_ref, w2_ref,
               b2_ref, ga_ref, be_ref, wp1_ref, cnt_ref, h_ref, fin_ref):
    cnt = jnp.sum(cntp_ref[...], axis=0)
    cntx = jnp.sum(cntxp_ref[...], axis=0)
    cnt_ref[...] = cnt

    ir = lax.broadcasted_iota(jnp.int32, (BN, 128), 1)
    oh0 = (x0_ref[...] == ir).astype(jnp.float32)
    oh1 = (x1_ref[...] == ir).astype(jnp.float32)
    h0 = jnp.dot(oh0, ne0_ref[...], preferred_element_type=jnp.float32)
    h0 = h0 + jnp.dot(oh1, ne1_ref[...], preferred_element_type=jnp.float32)
    fin = jnp.dot(h0, wp0_ref[...], preferred_element_type=jnp.float32) + bp_ref[...]

    agg = _cdot(cnt, e12_ref[...]) + _cdot(cntx, ex8_ref[...])
    hmid = jnp.dot(agg, w1_ref[...], preferred_element_type=jnp.float32)
    hmid = jnp.maximum(hmid + b1_ref[...], 0.0)
    h = jnp.dot(hmid, w2_ref[...], preferred_element_type=jnp.float32)
    h = ga_ref[...] * (h + b2_ref[...]) + be_ref[...]
    h = jnp.maximum(h, 0.0)
    for q in range(NQ):
        h_ref[q] = h[:, q * QW:(q + 1) * QW]
    fin_ref[...] = fin + jnp.dot(h, wp1_ref[...], preferred_element_type=jnp.float32)


_prep_call = pl.pallas_call(
    _prep_body,
    grid=(GRID,),
    in_specs=[
        pl.BlockSpec((NW, NCOL, BN), lambda i: (0, 0, i)),
        pl.BlockSpec((NW, XCOL, BN), lambda i: (0, 0, i)),
        pl.BlockSpec((BN, 1), lambda i: (i, 0)),
        pl.BlockSpec((BN, 1), lambda i: (i, 0)),
        pl.BlockSpec((128, EMB), lambda i: (0, 0)),
        pl.BlockSpec((128, EMB), lambda i: (0, 0)),
        pl.BlockSpec((EMB, EMB), lambda i: (0, 0)),
        pl.BlockSpec((1, EMB), lambda i: (0, 0)),
        pl.BlockSpec((NCOL, EMB), lambda i: (0, 0)),
        pl.BlockSpec((XCOL, EMB), lambda i: (0, 0)),
        pl.BlockSpec((EMB, 2 * EMB), lambda i: (0, 0)),
        pl.BlockSpec((1, 2 * EMB), lambda i: (0, 0)),
        pl.BlockSpec((2 * EMB, EMB), lambda i: (0, 0)),
        pl.BlockSpec((1, EMB), lambda i: (0, 0)),
        pl.BlockSpec((1, EMB), lambda i: (0, 0)),
        pl.BlockSpec((1, EMB), lambda i: (0, 0)),
        pl.BlockSpec((EMB, EMB), lambda i: (0, 0)),
    ],
    out_specs=[
        pl.BlockSpec((NCOL, BN), lambda i: (0, i)),
        pl.BlockSpec((NQ, BN, QW), lambda i: (0, i, 0)),
        pl.BlockSpec((BN, EMB), lambda i: (i, 0)),
    ],
    out_shape=[
        jax.ShapeDtypeStruct((NCOL, NP), jnp.float32),
        jax.ShapeDtypeStruct((NQ, N, QW), jnp.float32),
        jax.ShapeDtypeStruct((N, EMB), jnp.float32),
    ],
)


def _make_mlp_call(last: bool):
    def _mlp_body(p_ref, cnt_ref, e12_ref, w1_ref, b1_ref, w2_ref, b2_ref,
                  ga_ref, be_ref, wp_ref, fin_ref, h_ref, fino_ref):
        agg = jnp.concatenate([p_ref[q] for q in range(NQ)], axis=1)
        agg = agg + _cdot(cnt_ref[...], e12_ref[...])
        hmid = jnp.dot(agg, w1_ref[...], preferred_element_type=jnp.float32)
        hmid = jnp.maximum(hmid + b1_ref[...], 0.0)
        h = jnp.dot(hmid, w2_ref[...], preferred_element_type=jnp.float32)
        h = ga_ref[...] * (h + b2_ref[...]) + be_ref[...]
        if not last:
            h = jnp.maximum(h, 0.0)
        for q in range(NQ):
            h_ref[q] = h[:, q * QW:(q + 1) * QW]
        fino_ref[...] = fin_ref[...] + jnp.dot(h, wp_ref[...],
                                               preferred_element_type=jnp.float32)

    return pl.pallas_call(
        _mlp_body,
        grid=(GRID,),
        in_specs=[
            pl.BlockSpec((NQ, BN, QW), lambda i: (0, i, 0)),
            pl.BlockSpec((NCOL, BN), lambda i: (0, i)),
            pl.BlockSpec((NCOL, EMB), lambda i: (0, 0)),
            pl.BlockSpec((EMB, 2 * EMB), lambda i: (0, 0)),
            pl.BlockSpec((1, 2 * EMB), lambda i: (0, 0)),
            pl.BlockSpec((2 * EMB, EMB), lambda i: (0, 0)),
            pl.BlockSpec((1, EMB), lambda i: (0, 0)),
            pl.BlockSpec((1, EMB), lambda i: (0, 0)),
            pl.BlockSpec((1, EMB), lambda i: (0, 0)),
            pl.BlockSpec((EMB, EMB), lambda i: (0, 0)),
            pl.BlockSpec((BN, EMB), lambda i: (i, 0)),
        ],
        out_specs=[
            pl.BlockSpec((NQ, BN, QW), lambda i: (0, i, 0)),
            pl.BlockSpec((BN, EMB), lambda i: (i, 0)),
        ],
        out_shape=[
            jax.ShapeDtypeStruct((NQ, N, QW), jnp.float32),
            jax.ShapeDtypeStruct((N, EMB), jnp.float32),
        ],
    )


_mlp_mid = _make_mlp_call(last=False)
_mlp_last = _make_mlp_call(last=True)


def kernel(g, x, w, node_emb0, node_emb1, edge_emb0, edge_emb1,
           W1, b1, W2, b2, gamma, beta, Wp, bp):
    dst = g[1]
    wt = w.T
    w0 = wt[0]
    w1 = wt[1]
    sd = jnp.stack(
        [g[0].reshape(NS, NCH, CH), dst.reshape(NS, NCH, CH)], axis=2
    )

    ne0 = jnp.zeros((128, EMB), jnp.float32).at[: node_emb0.shape[0]].set(node_emb0)
    ne1 = jnp.zeros((128, EMB), jnp.float32).at[: node_emb1.shape[0]].set(node_emb1)

    e12 = jnp.zeros((L, NCOL, EMB), jnp.float32)
    e12 = e12.at[:, 0:6].set(edge_emb0)
    e12 = e12.at[:, 6:9].set(edge_emb1)

    ex8 = jnp.zeros((XCOL, EMB), jnp.float32)
    ex8 = ex8.at[0:3].set(node_emb0[:3])
    ex8 = ex8.at[4:7].set(node_emb1[:3])

    wp = Wp.reshape(L + 1, EMB, EMB)

    xc = (x[:, 0] + 4 * x[:, 1]).astype(jnp.int32)
    cntp = _count_kernel(dst, w0, w1)
    cntxp = _countx_kernel(dst, g[0], xc)

    cnt, h, fin = _prep_call(
        cntp.reshape(NW, NCOL, NP), cntxp.reshape(NW, XCOL, NP),
        x[:, 0:1], x[:, 1:2], ne0, ne1, wp[0], bp.reshape(1, EMB),
        e12[0], ex8,
        W1[0], b1[0].reshape(1, 2 * EMB), W2[0], b2[0].reshape(1, EMB),
        gamma[0].reshape(1, EMB), beta[0].reshape(1, EMB), wp[1],
    )

    for l in range(1, L):
        p = _seg_kernel(sd, h)
        call = _mlp_last if l == L - 1 else _mlp_mid
        h, fin = call(
            p, cnt, e12[l],
            W1[l], b1[l].reshape(1, 2 * EMB),
            W2[l], b2[l].reshape(1, EMB),
            gamma[l].reshape(1, EMB), beta[l].reshape(1, EMB),
            wp[l + 1], fin,
        )
    return fin


# speedup vs baseline: 2.1676x; 1.0182x over previous
import functools

import jax
import jax.numpy as jnp
from jax import lax
from jax.experimental import pallas as pl
from jax.experimental.pallas import tpu as pltpu
from jax.experimental.pallas import tpu_sc as plsc

N = 10000
E = 640000
EMB = 128
L = 5

NC = 2
NS = 16
NW = NC * NS
EPT = E // NW

NQ = 4
QW = EMB // NQ
EPS = E // NS
CH = 1000
NCH = EPS // CH
NP = 10240
RPT = NP // NS
SROW = 320

CCH = 800
NCCH = EPT // CCH
NCOL = 12

_mesh = plsc.VectorSubcoreMesh(
    core_axis_name="c", subcore_axis_name="s", num_cores=NC, num_subcores=NS
)

BN = 1280
GRID = NP // BN


@functools.partial(
    pl.kernel,
    out_type=jax.ShapeDtypeStruct((NW, NP * NCOL), jnp.float32),
    mesh=_mesh,
    scratch_types=[
        pltpu.VMEM((2, 3, CCH), jnp.int32),
        pltpu.VMEM((NP * NCOL,), jnp.float32),
        pltpu.SemaphoreType.DMA((2,)),
    ],
    compiler_params=pltpu.CompilerParams(needs_layout_passes=False),
)
def _count_kernel(dww_hbm, out_hbm, ev, cnt, esem):
    cid = lax.axis_index("c")
    sid = lax.axis_index("s")
    wid = cid * NS + sid

    pltpu.async_copy(dww_hbm.at[wid, 0], ev.at[0], esem.at[0])

    zeros16 = jnp.zeros((16,), jnp.float32)

    def _zero(i, _):
        cnt[pl.ds(i * 16, 16)] = zeros16
        return _

    lax.fori_loop(0, (NP * NCOL) // 16, _zero, None)

    ones16 = jnp.ones((16,), jnp.float32)

    def _chunk(c, _):
        b = lax.rem(c, 2)
        nb = lax.rem(c + 1, 2)
        pltpu.make_async_copy(dww_hbm.at[wid, c], ev.at[b], esem.at[b]).wait()

        @pl.when(c + 1 < NCCH)
        def _():
            pltpu.async_copy(dww_hbm.at[wid, c + 1], ev.at[nb], esem.at[nb])

        def _step(j, _):
            d = ev[b, 0, pl.ds(j * 16, 16)]
            a = ev[b, 1, pl.ds(j * 16, 16)]
            w = ev[b, 2, pl.ds(j * 16, 16)]
            plsc.addupdate_scatter(cnt, [a * NP + d], ones16)
            plsc.addupdate_scatter(cnt, [(w + 6) * NP + d], ones16)
            return _

        lax.fori_loop(0, CCH // 16, _step, None)
        return _

    lax.fori_loop(0, NCCH, _chunk, None)
    pltpu.sync_copy(cnt, out_hbm.at[wid])


XCOL = 8


@functools.partial(
    pl.kernel,
    out_type=jax.ShapeDtypeStruct((NW, NP * XCOL), jnp.float32),
    mesh=_mesh,
    scratch_types=[
        pltpu.VMEM((2, 2, CCH), jnp.int32),
        pltpu.VMEM((N,), jnp.int32),
        pltpu.VMEM((NP * XCOL,), jnp.float32),
        pltpu.SemaphoreType.DMA((2,)),
    ],
    compiler_params=pltpu.CompilerParams(needs_layout_passes=False),
)
def _countx_kernel(ds_hbm, xc_hbm, out_hbm, ev, xct, cnt, esem):
    cid = lax.axis_index("c")
    sid = lax.axis_index("s")
    wid = cid * NS + sid

    pltpu.async_copy(ds_hbm.at[wid, 0], ev.at[0], esem.at[0])
    pltpu.sync_copy(xc_hbm, xct)

    zeros16 = jnp.zeros((16,), jnp.float32)

    def _zero(i, _):
        cnt[pl.ds(i * 16, 16)] = zeros16
        return _

    lax.fori_loop(0, (NP * XCOL) // 16, _zero, None)

    ones16 = jnp.ones((16,), jnp.float32)

    def _chunk(c, _):
        b = lax.rem(c, 2)
        nb = lax.rem(c + 1, 2)
        pltpu.make_async_copy(ds_hbm.at[wid, c], ev.at[b], esem.at[b]).wait()

        @pl.when(c + 1 < NCCH)
        def _():
            pltpu.async_copy(ds_hbm.at[wid, c + 1], ev.at[nb], esem.at[nb])

        def _step(j, _):
            d = ev[b, 0, pl.ds(j * 16, 16)]
            s = ev[b, 1, pl.ds(j * 16, 16)]
            xcv = plsc.load_gather(xct, [s])
            x0 = xcv & 3
            x1 = xcv >> 2
            plsc.addupdate_scatter(cnt, [x0 * NP + d], ones16)
            plsc.addupdate_scatter(cnt, [(x1 + 4) * NP + d], ones16)
            return _

        lax.fori_loop(0, CCH // 16, _step, None)
        return _

    lax.fori_loop(0, NCCH, _chunk, None)
    pltpu.sync_copy(cnt, out_hbm.at[wid])


@functools.partial(
    pl.kernel,
    out_type=jax.ShapeDtypeStruct((NQ, NP, QW), jnp.float32),
    mesh=_mesh,
    scratch_types=[
        pltpu.VMEM((2, 2, CH), jnp.int32),
        pltpu.VMEM((2, CH, QW), jnp.float32),
        pltpu.VMEM((SROW, QW), jnp.float32),
        pltpu.VMEM_SHARED((NP, QW), jnp.float32),
        pltpu.SemaphoreType.DMA((2,)),
        pltpu.SemaphoreType.DMA((2,)),
    ],
    compiler_params=pltpu.CompilerParams(
        needs_layout_passes=False, use_tc_tiling_on_sc=False
    ),
)
def _seg_kernel(sd_hbm, h_hbm, out_hbm, cb, rows, stage, agg, gsem, ssem):
    cid = lax.axis_index("c")
    sid = lax.axis_index("s")

    zeros16 = jnp.zeros((16,), jnp.float32)

    def _zrow(i, _):
        r = i // (QW // 16)
        k = i % (QW // 16)
        stage[r, pl.ds(k * 16, 16)] = zeros16
        return _

    for p in range(NQ // NC):
        hq = h_hbm.at[NC * p + cid]

        lax.fori_loop(0, SROW * (QW // 16), _zrow, None)
        pltpu.sync_copy(stage, agg.at[pl.ds(sid * RPT, SROW)])
        pltpu.sync_copy(stage, agg.at[pl.ds(sid * RPT + SROW, SROW)])
        plsc.subcore_barrier()

        pltpu.sync_copy(sd_hbm.at[sid, 0], cb.at[0])
        pltpu.async_copy(hq.at[cb.at[0, 0]], rows.at[0], gsem.at[0])

        def _chunk(i, _):
            b = lax.rem(i, 2)
            nb = lax.rem(i + 1, 2)
            pltpu.make_async_copy(hq.at[cb.at[b, 0]], rows.at[b], gsem.at[b]).wait()

            @pl.when(i >= 1)
            def _():
                pltpu.make_async_copy(
                    rows.at[nb], agg.at[cb.at[nb, 1]], ssem.at[nb]
                ).wait()

            @pl.when(i + 1 < NCH)
            def _():
                pltpu.sync_copy(sd_hbm.at[sid, i + 1], cb.at[nb])
                pltpu.async_copy(hq.at[cb.at[nb, 0]], rows.at[nb], gsem.at[nb])

            pltpu.async_copy(rows.at[b], agg.at[cb.at[b, 1]], ssem.at[b], add=True)
            return _

        lax.fori_loop(0, NCH, _chunk, None)
        lb = (NCH - 1) % 2
        pltpu.make_async_copy(
            rows.at[lb], agg.at[cb.at[lb, 1]], ssem.at[lb]
        ).wait()
        plsc.subcore_barrier()

        pltpu.sync_copy(agg.at[pl.ds(sid * RPT, SROW)], stage)
        pltpu.sync_copy(stage, out_hbm.at[NC * p + cid, pl.ds(sid * RPT, SROW)])
        pltpu.sync_copy(agg.at[pl.ds(sid * RPT + SROW, SROW)], stage)
        pltpu.sync_copy(
            stage, out_hbm.at[NC * p + cid, pl.ds(sid * RPT + SROW, SROW)]
        )


def _cdot(cnt_t, tab):
    return lax.dot_general(
        cnt_t, tab, (((0,), (0,)), ((), ())),
        preferred_element_type=jnp.float32,
    )


def _prep_body(cntp_ref, cntxp_ref, x0_ref, x1_ref, ne0_ref, ne1_ref,
               wp0_ref, bp_ref, e12_ref, ex8_ref, w1_ref, b1_ref, w2_ref,
               b2_ref, ga_ref, be_ref, wp1_ref, cnt_ref, h_ref, fin_ref):
    cnt = jnp.sum(cntp_ref[...], axis=0)
    cntx = jnp.sum(cntxp_ref[...], axis=0)
    cnt_ref[...] = cnt

    ir = lax.broadcasted_iota(jnp.int32, (BN, 128), 1)
    oh0 = (x0_ref[...] == ir).astype(jnp.float32)
    oh1 = (x1_ref[...] == ir).astype(jnp.float32)
    h0 = jnp.dot(oh0, ne0_ref[...], preferred_element_type=jnp.float32)
    h0 = h0 + jnp.dot(oh1, ne1_ref[...], preferred_element_type=jnp.float32)
    fin = jnp.dot(h0, wp0_ref[...], preferred_element_type=jnp.float32) + bp_ref[...]

    agg = _cdot(cnt, e12_ref[...]) + _cdot(cntx, ex8_ref[...])
    hmid = jnp.dot(agg, w1_ref[...], preferred_element_type=jnp.float32)
    hmid = jnp.maximum(hmid + b1_ref[...], 0.0)
    h = jnp.dot(hmid, w2_ref[...], preferred_element_type=jnp.float32)
    h = ga_ref[...] * (h + b2_ref[...]) + be_ref[...]
    h = jnp.maximum(h, 0.0)
    for q in range(NQ):
        h_ref[q] = h[:, q * QW:(q + 1) * QW]
    fin_ref[...] = fin + jnp.dot(h, wp1_ref[...], preferred_element_type=jnp.float32)


_prep_call = pl.pallas_call(
    _prep_body,
    grid=(GRID,),
    in_specs=[
        pl.BlockSpec((NW, NCOL, BN), lambda i: (0, 0, i)),
        pl.BlockSpec((NW, XCOL, BN), lambda i: (0, 0, i)),
        pl.BlockSpec((BN, 1), lambda i: (i, 0)),
        pl.BlockSpec((BN, 1), lambda i: (i, 0)),
        pl.BlockSpec((128, EMB), lambda i: (0, 0)),
        pl.BlockSpec((128, EMB), lambda i: (0, 0)),
        pl.BlockSpec((EMB, EMB), lambda i: (0, 0)),
        pl.BlockSpec((1, EMB), lambda i: (0, 0)),
        pl.BlockSpec((NCOL, EMB), lambda i: (0, 0)),
        pl.BlockSpec((XCOL, EMB), lambda i: (0, 0)),
        pl.BlockSpec((EMB, 2 * EMB), lambda i: (0, 0)),
        pl.BlockSpec((1, 2 * EMB), lambda i: (0, 0)),
        pl.BlockSpec((2 * EMB, EMB), lambda i: (0, 0)),
        pl.BlockSpec((1, EMB), lambda i: (0, 0)),
        pl.BlockSpec((1, EMB), lambda i: (0, 0)),
        pl.BlockSpec((1, EMB), lambda i: (0, 0)),
        pl.BlockSpec((EMB, EMB), lambda i: (0, 0)),
    ],
    out_specs=[
        pl.BlockSpec((NCOL, BN), lambda i: (0, i)),
        pl.BlockSpec((NQ, BN, QW), lambda i: (0, i, 0)),
        pl.BlockSpec((BN, EMB), lambda i: (i, 0)),
    ],
    out_shape=[
        jax.ShapeDtypeStruct((NCOL, NP), jnp.float32),
        jax.ShapeDtypeStruct((NQ, N, QW), jnp.float32),
        jax.ShapeDtypeStruct((N, EMB), jnp.float32),
    ],
)


def _make_mlp_call(last: bool):
    def _mlp_body(p_ref, cnt_ref, e12_ref, w1_ref, b1_ref, w2_ref, b2_ref,
                  ga_ref, be_ref, wp_ref, fin_ref, h_ref, fino_ref):
        agg = jnp.concatenate([p_ref[q] for q in range(NQ)], axis=1)
        agg = agg + _cdot(cnt_ref[...], e12_ref[...])
        hmid = jnp.dot(agg, w1_ref[...], preferred_element_type=jnp.float32)
        hmid = jnp.maximum(hmid + b1_ref[...], 0.0)
        h = jnp.dot(hmid, w2_ref[...], preferred_element_type=jnp.float32)
        h = ga_ref[...] * (h + b2_ref[...]) + be_ref[...]
        if not last:
            h = jnp.maximum(h, 0.0)
        for q in range(NQ):
            h_ref[q] = h[:, q * QW:(q + 1) * QW]
        fino_ref[...] = fin_ref[...] + jnp.dot(h, wp_ref[...],
                                               preferred_element_type=jnp.float32)

    return pl.pallas_call(
        _mlp_body,
        grid=(GRID,),
        in_specs=[
            pl.BlockSpec((NQ, BN, QW), lambda i: (0, i, 0)),
            pl.BlockSpec((NCOL, BN), lambda i: (0, i)),
            pl.BlockSpec((NCOL, EMB), lambda i: (0, 0)),
            pl.BlockSpec((EMB, 2 * EMB), lambda i: (0, 0)),
            pl.BlockSpec((1, 2 * EMB), lambda i: (0, 0)),
            pl.BlockSpec((2 * EMB, EMB), lambda i: (0, 0)),
            pl.BlockSpec((1, EMB), lambda i: (0, 0)),
            pl.BlockSpec((1, EMB), lambda i: (0, 0)),
            pl.BlockSpec((1, EMB), lambda i: (0, 0)),
            pl.BlockSpec((EMB, EMB), lambda i: (0, 0)),
            pl.BlockSpec((BN, EMB), lambda i: (i, 0)),
        ],
        out_specs=[
            pl.BlockSpec((NQ, BN, QW), lambda i: (0, i, 0)),
            pl.BlockSpec((BN, EMB), lambda i: (i, 0)),
        ],
        out_shape=[
            jax.ShapeDtypeStruct((NQ, N, QW), jnp.float32),
            jax.ShapeDtypeStruct((N, EMB), jnp.float32),
        ],
    )


_mlp_mid = _make_mlp_call(last=False)
_mlp_last = _make_mlp_call(last=True)


def kernel(g, x, w, node_emb0, node_emb1, edge_emb0, edge_emb1,
           W1, b1, W2, b2, gamma, beta, Wp, bp):
    dst = g[1]
    wt = w.T
    w0 = wt[0]
    w1 = wt[1]
    sd = jnp.stack(
        [g[0].reshape(NS, NCH, CH), dst.reshape(NS, NCH, CH)], axis=2
    )

    ne0 = jnp.zeros((128, EMB), jnp.float32).at[: node_emb0.shape[0]].set(node_emb0)
    ne1 = jnp.zeros((128, EMB), jnp.float32).at[: node_emb1.shape[0]].set(node_emb1)

    e12 = jnp.zeros((L, NCOL, EMB), jnp.float32)
    e12 = e12.at[:, 0:6].set(edge_emb0)
    e12 = e12.at[:, 6:9].set(edge_emb1)

    ex8 = jnp.zeros((XCOL, EMB), jnp.float32)
    ex8 = ex8.at[0:3].set(node_emb0[:3])
    ex8 = ex8.at[4:7].set(node_emb1[:3])

    wp = Wp.reshape(L + 1, EMB, EMB)

    xc = (x[:, 0] + 4 * x[:, 1]).astype(jnp.int32)
    dww = jnp.stack(
        [dst.reshape(NW, NCCH, CCH), w0.reshape(NW, NCCH, CCH),
         w1.reshape(NW, NCCH, CCH)], axis=2
    )
    dsx = jnp.stack(
        [dst.reshape(NW, NCCH, CCH), g[0].reshape(NW, NCCH, CCH)], axis=2
    )
    cntp = _count_kernel(dww)
    cntxp = _countx_kernel(dsx, xc)

    cnt, h, fin = _prep_call(
        cntp.reshape(NW, NCOL, NP), cntxp.reshape(NW, XCOL, NP),
        x[:, 0:1], x[:, 1:2], ne0, ne1, wp[0], bp.reshape(1, EMB),
        e12[0], ex8,
        W1[0], b1[0].reshape(1, 2 * EMB), W2[0], b2[0].reshape(1, EMB),
        gamma[0].reshape(1, EMB), beta[0].reshape(1, EMB), wp[1],
    )

    for l in range(1, L):
        p = _seg_kernel(sd, h)
        call = _mlp_last if l == L - 1 else _mlp_mid
        h, fin = call(
            p, cnt, e12[l],
            W1[l], b1[l].reshape(1, 2 * EMB),
            W2[l], b2[l].reshape(1, EMB),
            gamma[l].reshape(1, EMB), beta[l].reshape(1, EMB),
            wp[l + 1], fin,
        )
    return fin
